# Initial kernel scaffold; baseline (speedup 1.0000x reference)
import jax
import jax.numpy as jnp
from jax.experimental import pallas as pl


def kernel(x, edge_index, W1, b1, W2, b2, Wfc, bfc):
    raise NotImplementedError("write your pallas kernel here")


# speedup vs baseline: 22.2090x; 22.2090x over previous
import functools

import jax
import jax.numpy as jnp
from jax import lax
from jax.experimental import pallas as pl
from jax.experimental.pallas import tpu as pltpu
from jax.experimental.pallas import tpu_sc as plsc

N = 10000
E = 320000
D = 128
D_OUT = 64

NC = 2
NS = 16
CH = 128
NCH = 80
PER_TILE = CH * NCH
EP = NC * NS * PER_TILE
NT = 10240
STRIPE = NT // NS
ROW_BLK = 1000

_mesh = plsc.VectorSubcoreMesh(core_axis_name="c", subcore_axis_name="s")


def _zero_f32_vec(ref, n):
    z = jnp.zeros((16,), jnp.float32)
    for k in range(n // 16):
        ref[pl.ds(16 * k, 16)] = z


@functools.partial(
    pl.kernel,
    mesh=_mesh,
    out_type=jax.ShapeDtypeStruct((NC, NT), jnp.float32),
    scratch_types=[
        pltpu.VMEM((NCH, CH), jnp.int32),
        pltpu.VMEM((CH,), jnp.float32),
        pltpu.VMEM((CH,), jnp.float32),
        pltpu.VMEM_SHARED((NT,), jnp.float32),
    ],
)
def _sc_degree(dstp_hbm, out_hbm, idx_v, ones_v, zero_v, acc):
    c = lax.axis_index("c")
    s = lax.axis_index("s")
    one = jnp.ones((16,), jnp.float32)
    for k in range(CH // 16):
        ones_v[pl.ds(16 * k, 16)] = one
    _zero_f32_vec(zero_v, CH)
    for k in range(STRIPE // CH):
        pltpu.sync_copy(zero_v, acc.at[pl.ds(s * STRIPE + k * CH, CH)])
    pltpu.sync_copy(dstp_hbm.at[c, s], idx_v)
    plsc.subcore_barrier()

    def body(j, carry):
        pltpu.sync_copy(ones_v, acc.at[idx_v.at[j]], add=True)
        return carry

    lax.fori_loop(0, NCH, body, 0)
    plsc.subcore_barrier()
    pltpu.sync_copy(acc.at[pl.ds(s * STRIPE, STRIPE)],
                    out_hbm.at[c, pl.ds(s * STRIPE, STRIPE)])


@functools.partial(
    pl.kernel,
    mesh=_mesh,
    out_type=jax.ShapeDtypeStruct((NC, NT, D), jnp.float32),
    scratch_types=[
        pltpu.VMEM((NCH, CH), jnp.int32),
        pltpu.VMEM((NCH, CH), jnp.int32),
        pltpu.VMEM((CH, D), jnp.float32),
        pltpu.VMEM_SHARED((NT, D), jnp.float32),
        pltpu.SemaphoreType.DMA,
    ],
)
def _sc_propagate(t_hbm, srcp_hbm, dstp_hbm, out_hbm, src_v, dst_v, rows_v, acc, sem):
    c = lax.axis_index("c")
    s = lax.axis_index("s")
    z = jnp.zeros((16,), jnp.float32)

    def zrow(r, carry):
        for k in range(D // 16):
            rows_v[r, pl.ds(16 * k, 16)] = z
        return carry

    lax.fori_loop(0, CH, zrow, 0)
    for k in range(STRIPE // CH):
        pltpu.sync_copy(rows_v, acc.at[pl.ds(s * STRIPE + k * CH, CH)])
    pltpu.sync_copy(srcp_hbm.at[c, s], src_v)
    pltpu.sync_copy(dstp_hbm.at[c, s], dst_v)
    plsc.subcore_barrier()

    def body(j, carry):
        pltpu.async_copy(t_hbm.at[src_v.at[j]], rows_v, sem).wait()
        pltpu.sync_copy(rows_v, acc.at[dst_v.at[j]], add=True)
        return carry

    lax.fori_loop(0, NCH, body, 0)
    plsc.subcore_barrier()
    pltpu.sync_copy(acc.at[pl.ds(s * STRIPE, STRIPE)],
                    out_hbm.at[c, pl.ds(s * STRIPE, STRIPE)])


def _tc_first(x_ref, w_ref, da_ref, db_ref, t_ref, dis_ref):
    deg = da_ref[...] + db_ref[...] + 1.0
    dis = lax.rsqrt(deg)
    h = jnp.dot(x_ref[...], w_ref[...], preferred_element_type=jnp.float32)
    t_ref[...] = h * dis
    dis_ref[...] = dis


def _tc_mid(pa_ref, pb_ref, t_ref, dis_ref, b_ref, w_ref, o_ref):
    dis = dis_ref[...]
    h = dis * (pa_ref[...] + pb_ref[...] + t_ref[...]) + b_ref[...]
    h = jnp.maximum(h, 0.0)
    o_ref[...] = jnp.dot(h, w_ref[...], preferred_element_type=jnp.float32) * dis


def _tc_last(pa_ref, pb_ref, t_ref, dis_ref, b_ref, w_ref, bfc_ref, o_ref):
    dis = dis_ref[...]
    h = dis * (pa_ref[...] + pb_ref[...] + t_ref[...]) + b_ref[...]
    o_ref[...] = (jnp.dot(h, w_ref[...], preferred_element_type=jnp.float32)
                  + bfc_ref[...])


_row_spec = pl.BlockSpec((ROW_BLK, D), lambda i: (i, 0))
_col_spec = pl.BlockSpec((ROW_BLK, 1), lambda i: (i, 0))
_w_spec = pl.BlockSpec((D, D), lambda i: (0, 0))
_b_spec = pl.BlockSpec((1, D), lambda i: (0, 0))
_GRID = (N // ROW_BLK,)


def _first_layer_pre(x, W1, da, db):
    return pl.pallas_call(
        _tc_first,
        grid=_GRID,
        in_specs=[_row_spec, _w_spec, _col_spec, _col_spec],
        out_specs=[_row_spec, _col_spec],
        out_shape=[jax.ShapeDtypeStruct((N, D), jnp.float32),
                   jax.ShapeDtypeStruct((N, 1), jnp.float32)],
    )(x, W1, da, db)


def _mid_layer(pa, pb, t, dis, b1, W2):
    return pl.pallas_call(
        _tc_mid,
        grid=_GRID,
        in_specs=[_row_spec, _row_spec, _row_spec, _col_spec, _b_spec, _w_spec],
        out_specs=_row_spec,
        out_shape=jax.ShapeDtypeStruct((N, D), jnp.float32),
    )(pa, pb, t, dis, b1, W2)


def _last_layer(pa, pb, t, dis, b2, Wfc, bfc):
    return pl.pallas_call(
        _tc_last,
        grid=_GRID,
        in_specs=[_row_spec, _row_spec, _row_spec, _col_spec, _b_spec,
                  pl.BlockSpec((D, D_OUT), lambda i: (0, 0)),
                  pl.BlockSpec((1, D_OUT), lambda i: (0, 0))],
        out_specs=pl.BlockSpec((ROW_BLK, D_OUT), lambda i: (i, 0)),
        out_shape=jax.ShapeDtypeStruct((N, D_OUT), jnp.float32),
    )(pa, pb, t, dis, b2, Wfc, bfc)


def kernel(x, edge_index, W1, b1, W2, b2, Wfc, bfc):
    pad = EP - E
    pad_src = (jnp.arange(pad, dtype=jnp.int32) * 37) % N
    pad_dst = N + (jnp.arange(pad, dtype=jnp.int32) % (NT - N))
    srcp = jnp.concatenate([edge_index[0], pad_src]).reshape(NC, NS, NCH, CH)
    dstp = jnp.concatenate([edge_index[1], pad_dst]).reshape(NC, NS, NCH, CH)

    degp = _sc_degree(dstp)
    da = degp[0, :N].reshape(N, 1)
    db = degp[1, :N].reshape(N, 1)

    t1, dis = _first_layer_pre(x, W1, da, db)
    p1 = _sc_propagate(t1, srcp, dstp)
    t2 = _mid_layer(p1[0, :N], p1[1, :N], t1, dis, b1.reshape(1, D), W2)
    p2 = _sc_propagate(t2, srcp, dstp)
    return _last_layer(p2[0, :N], p2[1, :N], t2, dis, b2.reshape(1, D),
                       Wfc, bfc.reshape(1, D_OUT))


# speedup vs baseline: 32.0158x; 1.4416x over previous
import functools

import jax
import jax.numpy as jnp
from jax import lax
from jax.experimental import pallas as pl
from jax.experimental.pallas import tpu as pltpu
from jax.experimental.pallas import tpu_sc as plsc

N = 10000
E = 320000
D = 128
D_OUT = 64

NC = 2
NS = 16
CH = 128
NCH = 80
PER_TILE = CH * NCH
EP = NC * NS * PER_TILE
NT = 10240
STRIPE = NT // NS
ROW_BLK = 1000

_mesh = plsc.VectorSubcoreMesh(core_axis_name="c", subcore_axis_name="s")


def _zero_f32_vec(ref, n):
    z = jnp.zeros((16,), jnp.float32)
    for k in range(n // 16):
        ref[pl.ds(16 * k, 16)] = z


@functools.partial(
    pl.kernel,
    mesh=_mesh,
    out_type=jax.ShapeDtypeStruct((NC, NT), jnp.float32),
    scratch_types=[
        pltpu.VMEM((NCH, CH), jnp.int32),
        pltpu.VMEM((CH,), jnp.float32),
        pltpu.VMEM((CH,), jnp.float32),
        pltpu.VMEM_SHARED((NT,), jnp.float32),
    ],
)
def _sc_degree(dstp_hbm, out_hbm, idx_v, ones_v, zero_v, acc):
    c = lax.axis_index("c")
    s = lax.axis_index("s")
    one = jnp.ones((16,), jnp.float32)
    for k in range(CH // 16):
        ones_v[pl.ds(16 * k, 16)] = one
    _zero_f32_vec(zero_v, CH)
    for k in range(STRIPE // CH):
        pltpu.sync_copy(zero_v, acc.at[pl.ds(s * STRIPE + k * CH, CH)])
    pltpu.sync_copy(dstp_hbm.at[c, s], idx_v)
    plsc.subcore_barrier()

    def body(j, carry):
        pltpu.sync_copy(ones_v, acc.at[idx_v.at[j]], add=True)
        return carry

    lax.fori_loop(0, NCH, body, 0)
    plsc.subcore_barrier()
    pltpu.sync_copy(acc.at[pl.ds(s * STRIPE, STRIPE)],
                    out_hbm.at[c, pl.ds(s * STRIPE, STRIPE)])


@functools.partial(
    pl.kernel,
    mesh=_mesh,
    out_type=jax.ShapeDtypeStruct((NC, NT, D), jnp.float32),
    scratch_types=[
        pltpu.VMEM((NCH, CH), jnp.int32),
        pltpu.VMEM((CH,), jnp.int32),
        pltpu.VMEM((CH,), jnp.int32),
        pltpu.VMEM((CH, D), jnp.float32),
        pltpu.VMEM((CH, D), jnp.float32),
        pltpu.VMEM_SHARED((NT, D), jnp.float32),
        pltpu.SemaphoreType.DMA,
        pltpu.SemaphoreType.DMA,
        pltpu.SemaphoreType.DMA,
        pltpu.SemaphoreType.DMA,
    ],
)
def _sc_propagate(t_hbm, srcp_hbm, dstp_hbm, out_hbm, src_v, dst_a, dst_b,
                  rows_a, rows_b, acc, sem_a, sem_b, sem_da, sem_db):
    c = lax.axis_index("c")
    s = lax.axis_index("s")
    z = jnp.zeros((16,), jnp.float32)

    def zrow(r, carry):
        for k in range(D // 16):
            rows_a[r, pl.ds(16 * k, 16)] = z
        return carry

    lax.fori_loop(0, CH, zrow, 0)
    for k in range(STRIPE // CH):
        pltpu.sync_copy(rows_a, acc.at[pl.ds(s * STRIPE + k * CH, CH)])
    pltpu.sync_copy(srcp_hbm.at[c, s], src_v)
    plsc.subcore_barrier()

    def gather(j, buf, sem):
        return pltpu.make_async_copy(t_hbm.at[src_v.at[j]], buf, sem)

    def dst_load(j, buf, sem):
        return pltpu.make_async_copy(dstp_hbm.at[c, s, j], buf, sem)

    gather(0, rows_a, sem_a).start()
    dst_load(0, dst_a, sem_da).start()

    def body(jj, carry):
        j0 = 2 * jj
        gather(j0 + 1, rows_b, sem_b).start()
        dst_load(j0 + 1, dst_b, sem_db).start()
        gather(j0, rows_a, sem_a).wait()
        dst_load(j0, dst_a, sem_da).wait()
        pltpu.sync_copy(rows_a, acc.at[dst_a], add=True)

        @pl.when(j0 + 2 < NCH)
        def _():
            gather(j0 + 2, rows_a, sem_a).start()
            dst_load(j0 + 2, dst_a, sem_da).start()

        gather(j0 + 1, rows_b, sem_b).wait()
        dst_load(j0 + 1, dst_b, sem_db).wait()
        pltpu.sync_copy(rows_b, acc.at[dst_b], add=True)
        return carry

    lax.fori_loop(0, NCH // 2, body, 0)
    plsc.subcore_barrier()
    pltpu.sync_copy(acc.at[pl.ds(s * STRIPE, STRIPE)],
                    out_hbm.at[c, pl.ds(s * STRIPE, STRIPE)])


def _tc_first(x_ref, w_ref, da_ref, db_ref, t_ref, dis_ref):
    deg = da_ref[...] + db_ref[...] + 1.0
    dis = lax.rsqrt(deg)
    h = jnp.dot(x_ref[...], w_ref[...], preferred_element_type=jnp.float32)
    t_ref[...] = h * dis
    dis_ref[...] = dis


def _tc_mid(pa_ref, pb_ref, t_ref, dis_ref, b_ref, w_ref, o_ref):
    dis = dis_ref[...]
    h = dis * (pa_ref[...] + pb_ref[...] + t_ref[...]) + b_ref[...]
    h = jnp.maximum(h, 0.0)
    o_ref[...] = jnp.dot(h, w_ref[...], preferred_element_type=jnp.float32) * dis


def _tc_last(pa_ref, pb_ref, t_ref, dis_ref, b_ref, w_ref, bfc_ref, o_ref):
    dis = dis_ref[...]
    h = dis * (pa_ref[...] + pb_ref[...] + t_ref[...]) + b_ref[...]
    o_ref[...] = (jnp.dot(h, w_ref[...], preferred_element_type=jnp.float32)
                  + bfc_ref[...])


_row_spec = pl.BlockSpec((ROW_BLK, D), lambda i: (i, 0))
_col_spec = pl.BlockSpec((ROW_BLK, 1), lambda i: (i, 0))
_w_spec = pl.BlockSpec((D, D), lambda i: (0, 0))
_b_spec = pl.BlockSpec((1, D), lambda i: (0, 0))
_GRID = (N // ROW_BLK,)


def _first_layer_pre(x, W1, da, db):
    return pl.pallas_call(
        _tc_first,
        grid=_GRID,
        in_specs=[_row_spec, _w_spec, _col_spec, _col_spec],
        out_specs=[_row_spec, _col_spec],
        out_shape=[jax.ShapeDtypeStruct((N, D), jnp.float32),
                   jax.ShapeDtypeStruct((N, 1), jnp.float32)],
    )(x, W1, da, db)


def _mid_layer(pa, pb, t, dis, b1, W2):
    return pl.pallas_call(
        _tc_mid,
        grid=_GRID,
        in_specs=[_row_spec, _row_spec, _row_spec, _col_spec, _b_spec, _w_spec],
        out_specs=_row_spec,
        out_shape=jax.ShapeDtypeStruct((N, D), jnp.float32),
    )(pa, pb, t, dis, b1, W2)


def _last_layer(pa, pb, t, dis, b2, Wfc, bfc):
    return pl.pallas_call(
        _tc_last,
        grid=_GRID,
        in_specs=[_row_spec, _row_spec, _row_spec, _col_spec, _b_spec,
                  pl.BlockSpec((D, D_OUT), lambda i: (0, 0)),
                  pl.BlockSpec((1, D_OUT), lambda i: (0, 0))],
        out_specs=pl.BlockSpec((ROW_BLK, D_OUT), lambda i: (i, 0)),
        out_shape=jax.ShapeDtypeStruct((N, D_OUT), jnp.float32),
    )(pa, pb, t, dis, b2, Wfc, bfc)


def kernel(x, edge_index, W1, b1, W2, b2, Wfc, bfc):
    pad = EP - E
    pad_src = (jnp.arange(pad, dtype=jnp.int32) * 37) % N
    pad_dst = N + (jnp.arange(pad, dtype=jnp.int32) % (NT - N))
    srcp = jnp.concatenate([edge_index[0], pad_src]).reshape(NC, NS, NCH, CH)
    dstp = jnp.concatenate([edge_index[1], pad_dst]).reshape(NC, NS, NCH, CH)

    degp = _sc_degree(dstp)
    da = degp[0, :N].reshape(N, 1)
    db = degp[1, :N].reshape(N, 1)

    t1, dis = _first_layer_pre(x, W1, da, db)
    p1 = _sc_propagate(t1, srcp, dstp)
    t2 = _mid_layer(p1[0, :N], p1[1, :N], t1, dis, b1.reshape(1, D), W2)
    p2 = _sc_propagate(t2, srcp, dstp)
    return _last_layer(p2[0, :N], p2[1, :N], t2, dis, b2.reshape(1, D),
                       Wfc, bfc.reshape(1, D_OUT))


# speedup vs baseline: 33.6301x; 1.0504x over previous
import functools

import jax
import jax.numpy as jnp
from jax import lax
from jax.experimental import pallas as pl
from jax.experimental.pallas import tpu as pltpu
from jax.experimental.pallas import tpu_sc as plsc

N = 10000
E = 320000
D = 128
D_OUT = 64

NC = 2
NS = 16
CH = 128
NCH = 80
PER_TILE = CH * NCH
EP = NC * NS * PER_TILE
NT = 10240
STRIPE = NT // NS
ROW_BLK = 1000

_mesh = plsc.VectorSubcoreMesh(core_axis_name="c", subcore_axis_name="s")


def _zero_f32_vec(ref, n):
    z = jnp.zeros((16,), jnp.float32)
    for k in range(n // 16):
        ref[pl.ds(16 * k, 16)] = z


@functools.partial(
    pl.kernel,
    mesh=_mesh,
    out_type=jax.ShapeDtypeStruct((NC, NT), jnp.float32),
    scratch_types=[
        pltpu.VMEM((NCH, CH), jnp.int32),
        pltpu.VMEM((CH,), jnp.float32),
        pltpu.VMEM((CH,), jnp.float32),
        pltpu.VMEM_SHARED((NT,), jnp.float32),
    ],
)
def _sc_degree(dstp_hbm, out_hbm, idx_v, ones_v, zero_v, acc):
    c = lax.axis_index("c")
    s = lax.axis_index("s")
    one = jnp.ones((16,), jnp.float32)
    for k in range(CH // 16):
        ones_v[pl.ds(16 * k, 16)] = one
    _zero_f32_vec(zero_v, CH)
    for k in range(STRIPE // CH):
        pltpu.sync_copy(zero_v, acc.at[pl.ds(s * STRIPE + k * CH, CH)])
    pltpu.sync_copy(dstp_hbm.at[c, s], idx_v)
    plsc.subcore_barrier()

    def body(j, carry):
        pltpu.sync_copy(ones_v, acc.at[idx_v.at[j]], add=True)
        return carry

    lax.fori_loop(0, NCH, body, 0)
    plsc.subcore_barrier()
    pltpu.sync_copy(acc.at[pl.ds(s * STRIPE, STRIPE)],
                    out_hbm.at[c, pl.ds(s * STRIPE, STRIPE)])


@functools.partial(
    pl.kernel,
    mesh=_mesh,
    out_type=jax.ShapeDtypeStruct((NC, NT, D), jnp.float32),
    scratch_types=[
        pltpu.VMEM((NCH, CH), jnp.int32),
        pltpu.VMEM((CH,), jnp.int32),
        pltpu.VMEM((CH,), jnp.int32),
        pltpu.VMEM((CH, D), jnp.float32),
        pltpu.VMEM((CH, D), jnp.float32),
        pltpu.VMEM_SHARED((NT, D), jnp.float32),
        pltpu.SemaphoreType.DMA,
        pltpu.SemaphoreType.DMA,
        pltpu.SemaphoreType.DMA,
        pltpu.SemaphoreType.DMA,
    ],
)
def _sc_propagate(t_hbm, srcp_hbm, dstp_hbm, out_hbm, src_v, dst_a, dst_b,
                  rows_a, rows_b, acc, sem_a, sem_b, sem_da, sem_db):
    c = lax.axis_index("c")
    s = lax.axis_index("s")
    z = jnp.zeros((16,), jnp.float32)

    def zrow(r, carry):
        for k in range(D // 16):
            rows_a[r, pl.ds(16 * k, 16)] = z
        return carry

    lax.fori_loop(0, CH, zrow, 0)
    for k in range(STRIPE // CH):
        pltpu.sync_copy(rows_a, acc.at[pl.ds(s * STRIPE + k * CH, CH)])
    pltpu.sync_copy(srcp_hbm.at[c, s], src_v)
    plsc.subcore_barrier()

    def gather(j, buf, sem):
        return pltpu.make_async_copy(t_hbm.at[src_v.at[j]], buf, sem)

    def dst_load(j, buf, sem):
        return pltpu.make_async_copy(dstp_hbm.at[c, s, j], buf, sem)

    gather(0, rows_a, sem_a).start()
    dst_load(0, dst_a, sem_da).start()

    def body(jj, carry):
        j0 = 2 * jj
        gather(j0 + 1, rows_b, sem_b).start()
        dst_load(j0 + 1, dst_b, sem_db).start()
        gather(j0, rows_a, sem_a).wait()
        dst_load(j0, dst_a, sem_da).wait()
        pltpu.sync_copy(rows_a, acc.at[dst_a], add=True)

        @pl.when(j0 + 2 < NCH)
        def _():
            gather(j0 + 2, rows_a, sem_a).start()
            dst_load(j0 + 2, dst_a, sem_da).start()

        gather(j0 + 1, rows_b, sem_b).wait()
        dst_load(j0 + 1, dst_b, sem_db).wait()
        pltpu.sync_copy(rows_b, acc.at[dst_b], add=True)
        return carry

    lax.fori_loop(0, NCH // 2, body, 0)
    plsc.subcore_barrier()
    pltpu.sync_copy(acc.at[pl.ds(s * STRIPE, STRIPE)],
                    out_hbm.at[c, pl.ds(s * STRIPE, STRIPE)])


def _tc_first(x_ref, w_ref, dg_ref, t_ref, dis_ref):
    deg = dg_ref[0] + dg_ref[1] + 1.0
    dis = lax.rsqrt(deg)
    h = jnp.dot(x_ref[...], w_ref[...], preferred_element_type=jnp.float32)
    t_ref[...] = h * dis
    dis_ref[...] = dis


def _tc_mid(pa_ref, pb_ref, t_ref, dis_ref, b_ref, w_ref, o_ref):
    dis = dis_ref[...]
    h = dis * (pa_ref[0] + pb_ref[0] + t_ref[...]) + b_ref[...]
    h = jnp.maximum(h, 0.0)
    o_ref[...] = jnp.dot(h, w_ref[...], preferred_element_type=jnp.float32) * dis


def _tc_last(pa_ref, pb_ref, t_ref, dis_ref, b_ref, w_ref, bfc_ref, o_ref):
    dis = dis_ref[...]
    h = dis * (pa_ref[0] + pb_ref[0] + t_ref[...]) + b_ref[...]
    o_ref[...] = (jnp.dot(h, w_ref[...], preferred_element_type=jnp.float32)
                  + bfc_ref[...])


_row_spec = pl.BlockSpec((ROW_BLK, D), lambda i: (i, 0))
_col_spec = pl.BlockSpec((ROW_BLK, 1), lambda i: (i, 0))
_w_spec = pl.BlockSpec((D, D), lambda i: (0, 0))
_b_spec = pl.BlockSpec((1, D), lambda i: (0, 0))
_pa_spec = pl.BlockSpec((1, ROW_BLK, D), lambda i: (0, i, 0))
_pb_spec = pl.BlockSpec((1, ROW_BLK, D), lambda i: (1, i, 0))
_dg_spec = pl.BlockSpec((2, ROW_BLK, 1), lambda i: (0, i, 0))
_GRID = (N // ROW_BLK,)


def _first_layer_pre(x, W1, degp):
    return pl.pallas_call(
        _tc_first,
        grid=_GRID,
        in_specs=[_row_spec, _w_spec, _dg_spec],
        out_specs=[_row_spec, _col_spec],
        out_shape=[jax.ShapeDtypeStruct((N, D), jnp.float32),
                   jax.ShapeDtypeStruct((N, 1), jnp.float32)],
    )(x, W1, degp)


def _mid_layer(p, t, dis, b1, W2):
    return pl.pallas_call(
        _tc_mid,
        grid=_GRID,
        in_specs=[_pa_spec, _pb_spec,
                  _row_spec, _col_spec, _b_spec, _w_spec],
        out_specs=_row_spec,
        out_shape=jax.ShapeDtypeStruct((N, D), jnp.float32),
    )(p, p, t, dis, b1, W2)


def _last_layer(p, t, dis, b2, Wfc, bfc):
    return pl.pallas_call(
        _tc_last,
        grid=_GRID,
        in_specs=[_pa_spec, _pb_spec,
                  _row_spec, _col_spec, _b_spec,
                  pl.BlockSpec((D, D_OUT), lambda i: (0, 0)),
                  pl.BlockSpec((1, D_OUT), lambda i: (0, 0))],
        out_specs=pl.BlockSpec((ROW_BLK, D_OUT), lambda i: (i, 0)),
        out_shape=jax.ShapeDtypeStruct((N, D_OUT), jnp.float32),
    )(p, p, t, dis, b2, Wfc, bfc)


def kernel(x, edge_index, W1, b1, W2, b2, Wfc, bfc):
    pad = EP - E
    pad_src = (jnp.arange(pad, dtype=jnp.int32) * 37) % N
    pad_dst = N + (jnp.arange(pad, dtype=jnp.int32) % (NT - N))
    srcp = jnp.concatenate([edge_index[0], pad_src]).reshape(NC, NS, NCH, CH)
    dstp = jnp.concatenate([edge_index[1], pad_dst]).reshape(NC, NS, NCH, CH)

    degp = _sc_degree(dstp).reshape(NC, NT, 1)
    t1, dis = _first_layer_pre(x, W1, degp)
    p1 = _sc_propagate(t1, srcp, dstp)
    t2 = _mid_layer(p1, t1, dis, b1.reshape(1, D), W2)
    p2 = _sc_propagate(t2, srcp, dstp)
    return _last_layer(p2, t2, dis, b2.reshape(1, D),
                       Wfc, bfc.reshape(1, D_OUT))
